# baseline (speedup 1.0000x reference)
import functools

import jax
import jax.numpy as jnp
from jax.experimental import pallas as pl

_K = 10
_IOU_FACTOR_IS_6 = True
_EPS = 1e-9

_ATAN_C = (0.9999994161532371, -0.33330223018999205, 0.19951119254101668,
           -0.13933275185449312, 0.09709477935292922, -0.05688276598949693,
           0.022568219028796452, -0.004257820308310879)
_HALF_PI = 1.5707963267948966


def _atan_pos(x):
    inv = x > 1.0
    t = jnp.where(inv, 1.0 / x, x)
    t2 = t * t
    p = _ATAN_C[7]
    for c in (_ATAN_C[6], _ATAN_C[5], _ATAN_C[4], _ATAN_C[3], _ATAN_C[2],
              _ATAN_C[1], _ATAN_C[0]):
        p = p * t2 + c
    p = p * t
    return jnp.where(inv, _HALF_PI - p, p)


def _ciou(ax1, ay1, ax2, ay2, bx1, by1, bx2, by2):
    ix1 = jnp.maximum(ax1, bx1)
    iy1 = jnp.maximum(ay1, by1)
    ix2 = jnp.minimum(ax2, bx2)
    iy2 = jnp.minimum(ay2, by2)
    inter = jnp.maximum(ix2 - ix1, 0.0) * jnp.maximum(iy2 - iy1, 0.0)
    area_a = (ax2 - ax1) * (ay2 - ay1)
    area_b = (bx2 - bx1) * (by2 - by1)
    union = area_a + area_b - inter + _EPS
    iou = inter / union
    cw = jnp.maximum(ax2, bx2) - jnp.minimum(ax1, bx1)
    ch = jnp.maximum(ay2, by2) - jnp.minimum(ay1, by1)
    c2 = cw * cw + ch * ch + _EPS
    dx = ax1 + ax2 - bx1 - bx2
    dy = ay1 + ay2 - by1 - by2
    rho2 = (dx * dx + dy * dy) * 0.25
    w1 = ax2 - ax1 + _EPS
    h1 = ay2 - ay1 + _EPS
    w2 = bx2 - bx1 + _EPS
    h2 = by2 - by1 + _EPS
    datan = _atan_pos(w1 / h1) - _atan_pos(w2 / h2)
    v = (4.0 / (jnp.pi * jnp.pi)) * datan * datan
    alpha = v / (1.0 - iou + v + _EPS)
    return iou - rho2 / c2 - alpha * v


def _body(pcls_ref, panc_ref, pbbox_ref, tgt_ref, ancsc_ref, out_ref,
          *, n_cls, n_tgt, n_bins):
    f32 = jnp.float32
    X = pcls_ref[0]
    ANC = panc_ref[0]
    PB = pbbox_ref[0]
    TGT = tgt_ref[0]
    AS = ancsc_ref[...]

    tcls = jnp.maximum(TGT[:, 0:1], 0.0).astype(jnp.int32)
    tx1 = TGT[:, 1:2]
    ty1 = TGT[:, 2:3]
    tx2 = TGT[:, 3:4]
    ty2 = TGT[:, 4:5]

    ax = AS[0:1, :]
    ay = AS[1:2, :]
    sc = AS[2:3, :]

    px1 = PB[0:1, :]
    py1 = PB[1:2, :]
    px2 = PB[2:3, :]
    py2 = PB[3:4, :]

    grid = ((tx1 < ax) & (ax < tx2) & (ty1 < ay) & (ay < ty2)).astype(f32)
    iou_mat = jnp.clip(
        _ciou(tx1, ty1, tx2, ty2, px1, py1, px2, py2), 0.0, 1.0)

    c_iota = jax.lax.broadcasted_iota(jnp.int32, (n_tgt, n_cls), 1)
    E = (c_iota == tcls).astype(f32)
    S = 0.5 * jnp.tanh(0.5 * X) + 0.5
    cls_mat = jax.lax.dot(E, S, preferred_element_type=f32)
    logit_mat = jax.lax.dot(E, X, preferred_element_type=f32)

    i2 = iou_mat * iou_mat
    tm = grid * (i2 * i2 * i2) * jnp.sqrt(cls_mat)

    mi = jnp.max(iou_mat, axis=1, keepdims=True)

    n_a = tm.shape[1]
    mreg = [jnp.full((n_tgt, 128), -1.0, f32) for _ in range(_K)]
    for cidx in range(n_a // 128):
        x = tm[:, cidx * 128:(cidx + 1) * 128]
        for k in range(_K):
            hi = jnp.maximum(mreg[k], x)
            x = jnp.minimum(mreg[k], x)
            mreg[k] = hi
    mt = None
    for rnd in range(_K - 1):
        cur = mreg[0]
        for k in range(1, _K):
            cur = jnp.maximum(cur, mreg[k])
        rm = jnp.max(cur, axis=1, keepdims=True)
        if rnd == 0:
            mt = rm
        for k in range(_K):
            mreg[k] = jnp.where(mreg[k] >= rm, -1.0, mreg[k])
    cur = mreg[0]
    for k in range(1, _K):
        cur = jnp.maximum(cur, mreg[k])
    thr = jnp.max(cur, axis=1, keepdims=True)

    tt = jnp.where((tm >= thr) & (tm > 0.0), tm, 0.0)

    n_topk = jnp.sum((tt > 0.0).astype(f32), axis=0, keepdims=True)
    n_grid = jnp.sum(grid, axis=0, keepdims=True)
    valid = ((n_grid * n_topk) > 0.0).astype(f32)

    colmax = jnp.max(tt, axis=0, keepdims=True)
    t_iota = jax.lax.broadcasted_iota(jnp.int32, (n_tgt, 1), 0)
    mclm = tt == colmax
    u = jnp.min(jnp.where(mclm, t_iota, n_tgt), axis=0, keepdims=True)
    fsel = (t_iota == u).astype(f32)

    norm = jnp.sum(fsel * tm * (mi / (mt + _EPS)), axis=0, keepdims=True)
    s = norm * valid

    abx1 = jnp.sum(fsel * tx1, axis=0, keepdims=True)
    aby1 = jnp.sum(fsel * ty1, axis=0, keepdims=True)
    abx2 = jnp.sum(fsel * tx2, axis=0, keepdims=True)
    aby2 = jnp.sum(fsel * ty2, axis=0, keepdims=True)

    p_s = jnp.sum(s)
    p_gather = jnp.sum(fsel * logit_mat * s)

    p_bce = jnp.sum(jnp.maximum(X, 0.0) + jnp.log1p(jnp.exp(-jnp.abs(X))))

    inv_sc = 1.0 / sc
    iou_l = _ciou(px1 * inv_sc, py1 * inv_sc, px2 * inv_sc, py2 * inv_sc,
                  abx1 * inv_sc, aby1 * inv_sc, abx2 * inv_sc, aby2 * inv_sc)
    p_iou = jnp.sum((1.0 - iou_l) * s)

    axn = ax * inv_sc
    ayn = ay * inv_sc
    dmax = float(n_bins) - 1.01
    d0 = jnp.clip(axn - abx1 * inv_sc, 0.0, dmax)
    d1 = jnp.clip(ayn - aby1 * inv_sc, 0.0, dmax)
    d2 = jnp.clip(abx2 * inv_sc - axn, 0.0, dmax)
    d3 = jnp.clip(aby2 * inv_sc - ayn, 0.0, dmax)
    r_iota = jax.lax.broadcasted_iota(
        jnp.int32, (n_bins, 1), 0).astype(f32)
    dfl = jnp.zeros_like(axn)
    for j, dj in enumerate((d0, d1, d2, d3)):
        xj = ANC[j * n_bins:(j + 1) * n_bins, :]
        mj = jnp.max(xj, axis=0, keepdims=True)
        lse = mj + jnp.log(jnp.sum(jnp.exp(xj - mj), axis=0, keepdims=True))
        hat = jnp.maximum(1.0 - jnp.abs(r_iota - dj), 0.0)
        interp = jnp.sum(xj * hat, axis=0, keepdims=True)
        dfl = dfl + (lse - interp)
    p_dfl = jnp.sum(0.25 * dfl * s)

    li = jax.lax.broadcasted_iota(jnp.int32, (1, 128), 1)
    row = (jnp.where(li == 0, p_s, 0.0) + jnp.where(li == 1, p_bce, 0.0)
           + jnp.where(li == 2, p_gather, 0.0) + jnp.where(li == 3, p_iou, 0.0)
           + jnp.where(li == 4, p_dfl, 0.0))
    out_ref[0] = jnp.broadcast_to(row, (8, 128))


@jax.jit
def kernel(predicts_cls, predicts_anc, predicts_bbox, targets, anchors,
           scaler):
    b, a, c = predicts_cls.shape
    r = predicts_anc.shape[-1]
    t = targets.shape[1]

    ap = ((a + 127) // 128) * 128
    pad = ap - a
    pcls_t = jnp.pad(jnp.transpose(predicts_cls, (0, 2, 1)),
                     ((0, 0), (0, 0), (0, pad)), constant_values=-1e9)
    panc_t = jnp.pad(
        jnp.transpose(predicts_anc.reshape(b, a, 4 * r), (0, 2, 1)),
        ((0, 0), (0, 0), (0, pad)))
    pbbox_t = jnp.pad(jnp.transpose(predicts_bbox, (0, 2, 1)),
                      ((0, 0), (0, 0), (0, pad)))
    ancsc = jnp.concatenate(
        [jnp.pad(anchors.T, ((0, 0), (0, pad)), constant_values=-1e6),
         jnp.pad(scaler[None, :], ((0, 0), (0, pad)), constant_values=1.0)],
        axis=0)

    body = functools.partial(_body, n_cls=c, n_tgt=t, n_bins=r)
    parts = pl.pallas_call(
        body,
        grid=(b,),
        in_specs=[
            pl.BlockSpec((1, c, ap), lambda i: (i, 0, 0)),
            pl.BlockSpec((1, 4 * r, ap), lambda i: (i, 0, 0)),
            pl.BlockSpec((1, 4, ap), lambda i: (i, 0, 0)),
            pl.BlockSpec((1, t, 5), lambda i: (i, 0, 0)),
            pl.BlockSpec((3, ap), lambda i: (0, 0)),
        ],
        out_specs=pl.BlockSpec((1, 8, 128), lambda i: (i, 0, 0)),
        out_shape=jax.ShapeDtypeStruct((b, 8, 128), jnp.float32),
    )(pcls_t, panc_t, pbbox_t, targets, ancsc)

    p = jnp.sum(parts[:, 0, :], axis=0)
    cls_norm = jnp.maximum(p[0], 1.0)
    loss_cls = (p[1] - p[2]) / cls_norm
    loss_iou = p[3] / cls_norm
    loss_dfl = p[4] / cls_norm
    return 0.5 * loss_iou + 7.5 * loss_dfl + 1.5 * loss_cls


# speedup vs baseline: 1.2931x; 1.2931x over previous
import functools

import jax
import jax.numpy as jnp
from jax.experimental import pallas as pl

_K = 10
_IOU_FACTOR_IS_6 = True
_EPS = 1e-9

_ATAN_C = (0.9999994161532371, -0.33330223018999205, 0.19951119254101668,
           -0.13933275185449312, 0.09709477935292922, -0.05688276598949693,
           0.022568219028796452, -0.004257820308310879)
_HALF_PI = 1.5707963267948966


def _atan_pos(x):
    inv = x > 1.0
    t = jnp.where(inv, 1.0 / x, x)
    t2 = t * t
    p = _ATAN_C[7]
    for c in (_ATAN_C[6], _ATAN_C[5], _ATAN_C[4], _ATAN_C[3], _ATAN_C[2],
              _ATAN_C[1], _ATAN_C[0]):
        p = p * t2 + c
    p = p * t
    return jnp.where(inv, _HALF_PI - p, p)


def _ciou(ax1, ay1, ax2, ay2, bx1, by1, bx2, by2):
    ix1 = jnp.maximum(ax1, bx1)
    iy1 = jnp.maximum(ay1, by1)
    ix2 = jnp.minimum(ax2, bx2)
    iy2 = jnp.minimum(ay2, by2)
    inter = jnp.maximum(ix2 - ix1, 0.0) * jnp.maximum(iy2 - iy1, 0.0)
    area_a = (ax2 - ax1) * (ay2 - ay1)
    area_b = (bx2 - bx1) * (by2 - by1)
    union = area_a + area_b - inter + _EPS
    iou = inter / union
    cw = jnp.maximum(ax2, bx2) - jnp.minimum(ax1, bx1)
    ch = jnp.maximum(ay2, by2) - jnp.minimum(ay1, by1)
    c2 = cw * cw + ch * ch + _EPS
    dx = ax1 + ax2 - bx1 - bx2
    dy = ay1 + ay2 - by1 - by2
    rho2 = (dx * dx + dy * dy) * 0.25
    w1 = ax2 - ax1 + _EPS
    h1 = ay2 - ay1 + _EPS
    w2 = bx2 - bx1 + _EPS
    h2 = by2 - by1 + _EPS
    datan = _atan_pos(w1 / h1) - _atan_pos(w2 / h2)
    v = (4.0 / (jnp.pi * jnp.pi)) * datan * datan
    alpha = v / (1.0 - iou + v + _EPS)
    return iou - rho2 / c2 - alpha * v


def _body(pcls_ref, panc_ref, pbbox_ref, tgt_ref, ancsc_ref, out_ref,
          *, n_cls, n_tgt, n_bins):
    f32 = jnp.float32
    X = pcls_ref[0]
    ANC = panc_ref[0]
    PB = pbbox_ref[0]
    TGT = tgt_ref[0]
    AS = ancsc_ref[...]

    tcls = jnp.maximum(TGT[:, 0:1], 0.0).astype(jnp.int32)
    tx1 = TGT[:, 1:2]
    ty1 = TGT[:, 2:3]
    tx2 = TGT[:, 3:4]
    ty2 = TGT[:, 4:5]

    ax = AS[0:1, :]
    ay = AS[1:2, :]
    sc = AS[2:3, :]

    px1 = PB[0:1, :]
    py1 = PB[1:2, :]
    px2 = PB[2:3, :]
    py2 = PB[3:4, :]

    grid = ((tx1 < ax) & (ax < tx2) & (ty1 < ay) & (ay < ty2)).astype(f32)
    iou_mat = jnp.clip(
        _ciou(tx1, ty1, tx2, ty2, px1, py1, px2, py2), 0.0, 1.0)

    c_iota = jax.lax.broadcasted_iota(jnp.int32, (n_tgt, n_cls), 1)
    E = (c_iota == tcls).astype(f32)
    S = 0.5 * jnp.tanh(0.5 * X) + 0.5
    cls_mat = jax.lax.dot(E, S, preferred_element_type=f32)
    logit_mat = jax.lax.dot(E, X, preferred_element_type=f32)

    i2 = iou_mat * iou_mat
    tm = grid * (i2 * i2 * i2) * jnp.sqrt(cls_mat)

    mi = jnp.max(iou_mat, axis=1, keepdims=True)

    n_a = tm.shape[1]
    lpad = (-n_a) % 128
    tm_p = jnp.concatenate([tm, jnp.zeros((n_tgt, lpad), f32)], axis=1)
    mreg = [jnp.full((n_tgt, 128), -1.0, f32) for _ in range(_K)]
    for cidx in range((n_a + lpad) // 128):
        x = tm_p[:, cidx * 128:(cidx + 1) * 128]
        for k in range(_K):
            hi = jnp.maximum(mreg[k], x)
            x = jnp.minimum(mreg[k], x)
            mreg[k] = hi
    mt = None
    for rnd in range(_K - 1):
        cur = mreg[0]
        for k in range(1, _K):
            cur = jnp.maximum(cur, mreg[k])
        rm = jnp.max(cur, axis=1, keepdims=True)
        if rnd == 0:
            mt = rm
        for k in range(_K):
            mreg[k] = jnp.where(mreg[k] >= rm, -1.0, mreg[k])
    cur = mreg[0]
    for k in range(1, _K):
        cur = jnp.maximum(cur, mreg[k])
    thr = jnp.max(cur, axis=1, keepdims=True)

    tt = jnp.where((tm >= thr) & (tm > 0.0), tm, 0.0)

    n_topk = jnp.sum((tt > 0.0).astype(f32), axis=0, keepdims=True)
    n_grid = jnp.sum(grid, axis=0, keepdims=True)
    valid = ((n_grid * n_topk) > 0.0).astype(f32)

    colmax = jnp.max(tt, axis=0, keepdims=True)
    t_iota = jax.lax.broadcasted_iota(jnp.int32, (n_tgt, 1), 0)
    mclm = tt == colmax
    u = jnp.min(jnp.where(mclm, t_iota, n_tgt), axis=0, keepdims=True)
    fsel = (t_iota == u).astype(f32)

    norm = jnp.sum(fsel * tm * (mi / (mt + _EPS)), axis=0, keepdims=True)
    s = norm * valid

    abx1 = jnp.sum(fsel * tx1, axis=0, keepdims=True)
    aby1 = jnp.sum(fsel * ty1, axis=0, keepdims=True)
    abx2 = jnp.sum(fsel * tx2, axis=0, keepdims=True)
    aby2 = jnp.sum(fsel * ty2, axis=0, keepdims=True)

    p_s = jnp.sum(s)
    p_gather = jnp.sum(fsel * logit_mat * s)

    p_bce = jnp.sum(jnp.maximum(X, 0.0) + jnp.log1p(jnp.exp(-jnp.abs(X))))

    inv_sc = 1.0 / sc
    iou_l = _ciou(px1 * inv_sc, py1 * inv_sc, px2 * inv_sc, py2 * inv_sc,
                  abx1 * inv_sc, aby1 * inv_sc, abx2 * inv_sc, aby2 * inv_sc)
    p_iou = jnp.sum((1.0 - iou_l) * s)

    axn = ax * inv_sc
    ayn = ay * inv_sc
    dmax = float(n_bins) - 1.01
    d0 = jnp.clip(axn - abx1 * inv_sc, 0.0, dmax)
    d1 = jnp.clip(ayn - aby1 * inv_sc, 0.0, dmax)
    d2 = jnp.clip(abx2 * inv_sc - axn, 0.0, dmax)
    d3 = jnp.clip(aby2 * inv_sc - ayn, 0.0, dmax)
    r_iota = jax.lax.broadcasted_iota(
        jnp.int32, (n_bins, 1), 0).astype(f32)
    dfl = jnp.zeros_like(axn)
    for j, dj in enumerate((d0, d1, d2, d3)):
        xj = ANC[j * n_bins:(j + 1) * n_bins, :]
        mj = jnp.max(xj, axis=0, keepdims=True)
        lse = mj + jnp.log(jnp.sum(jnp.exp(xj - mj), axis=0, keepdims=True))
        hat = jnp.maximum(1.0 - jnp.abs(r_iota - dj), 0.0)
        interp = jnp.sum(xj * hat, axis=0, keepdims=True)
        dfl = dfl + (lse - interp)
    p_dfl = jnp.sum(0.25 * dfl * s)

    li = jax.lax.broadcasted_iota(jnp.int32, (1, 128), 1)
    row = (jnp.where(li == 0, p_s, 0.0) + jnp.where(li == 1, p_bce, 0.0)
           + jnp.where(li == 2, p_gather, 0.0) + jnp.where(li == 3, p_iou, 0.0)
           + jnp.where(li == 4, p_dfl, 0.0))
    out_ref[0] = jnp.broadcast_to(row, (8, 128))


@jax.jit
def kernel(predicts_cls, predicts_anc, predicts_bbox, targets, anchors,
           scaler):
    b, a, c = predicts_cls.shape
    r = predicts_anc.shape[-1]
    t = targets.shape[1]

    pcls_t = jnp.transpose(predicts_cls, (0, 2, 1))
    panc_t = jnp.transpose(predicts_anc.reshape(b, a, 4 * r), (0, 2, 1))
    pbbox_t = jnp.transpose(predicts_bbox, (0, 2, 1))
    ancsc = jnp.concatenate([anchors.T, scaler[None, :]], axis=0)

    body = functools.partial(_body, n_cls=c, n_tgt=t, n_bins=r)
    parts = pl.pallas_call(
        body,
        grid=(b,),
        in_specs=[
            pl.BlockSpec((1, c, a), lambda i: (i, 0, 0)),
            pl.BlockSpec((1, 4 * r, a), lambda i: (i, 0, 0)),
            pl.BlockSpec((1, 4, a), lambda i: (i, 0, 0)),
            pl.BlockSpec((1, t, 5), lambda i: (i, 0, 0)),
            pl.BlockSpec((3, a), lambda i: (0, 0)),
        ],
        out_specs=pl.BlockSpec((1, 8, 128), lambda i: (i, 0, 0)),
        out_shape=jax.ShapeDtypeStruct((b, 8, 128), jnp.float32),
    )(pcls_t, panc_t, pbbox_t, targets, ancsc)

    p = jnp.sum(parts[:, 0, :], axis=0)
    cls_norm = jnp.maximum(p[0], 1.0)
    loss_cls = (p[1] - p[2]) / cls_norm
    loss_iou = p[3] / cls_norm
    loss_dfl = p[4] / cls_norm
    return 0.5 * loss_iou + 7.5 * loss_dfl + 1.5 * loss_cls
